# baseline (speedup 1.0000x reference)
import functools

import jax
import jax.numpy as jnp
from jax import lax
from jax.experimental import pallas as pl
from jax.experimental.pallas import tpu as pltpu
from jax.experimental.pallas import tpu_sc as plsc

STRIDE9 = 9
L = 16
NT = 32
CCH = 16


def _sc_gather_body(B, C, H, W, M,
                    kpts_hbm, table_hbm, taps_hbm,
                    kv, idx2d, data, tap, tapT, sem):
    HW = H * W
    RPP = HW // L
    ci = lax.axis_index("cx")
    si = lax.axis_index("sx")
    wid = si * 2 + ci
    b = wid // (NT // B)

    pltpu.sync_copy(kpts_hbm.at[pl.ds(wid * 2 * L, 2 * L)], kv)

    lanes = lax.iota(jnp.int32, L)
    kx = plsc.load_gather(kv, [lanes * 2])
    ky = plsc.load_gather(kv, [lanes * 2 + 1])
    x = (kx * float(H)).astype(jnp.int32)
    y = (ky * float(W)).astype(jnp.int32)
    pbase = b * (C * RPP)

    def row_lane(p):
        h = p // W
        w = p - h * W
        row = ((lax.shift_right_logical(h, 3) * (W // 128)
                + lax.shift_right_logical(w, 7)) * 64
               + lax.bitwise_and(h, 7) * 8
               + lax.shift_right_logical(lax.bitwise_and(w, 127), 4))
        return row, lax.bitwise_and(w, 15)

    for k in range(9):
        dx = (k // 3 - 1) * STRIDE9
        dy = (k % 3 - 1) * STRIDE9
        pk = jnp.clip((x + dx) * W + (y + dy), 0, HW - 1)
        rk, lk = row_lane(pk)

        def ibody(cc, _):
            cplane = pbase + cc * RPP
            vk = jnp.full((L,), cplane, jnp.int32) + rk
            idx2d[lax.shift_right_logical(cc, 3),
                  pl.ds(lax.bitwise_and(cc, 7) * L, L)] = vk
            return 0
        lax.fori_loop(0, C, ibody, 0)

        cps = [pltpu.async_copy(table_hbm.at[idx2d.at[j]],
                                data.at[pl.ds(j * 128, 128)], sem)
               for j in range(8)]
        for cp in cps:
            cp.wait()

        def ebody(cc, _):
            rows = cc * L + lanes
            val = plsc.load_gather(data, [rows, lk])
            tap[cc, :] = val
            return 0
        lax.fori_loop(0, C, ebody, 0)

        def tbody(c, _):
            cvec = jnp.full((L,), c, jnp.int32)
            plsc.store_scatter(tapT, [lanes, cvec], tap[c, :])
            return 0
        lax.fori_loop(0, C, tbody, 0)
        pltpu.sync_copy(tapT, taps_hbm.at[k * NT + wid])


def _sc_gather(kpts_flat, table, B, C, H, W, M):
    mesh = plsc.VectorSubcoreMesh(core_axis_name="cx", subcore_axis_name="sx")
    body = functools.partial(_sc_gather_body, B, C, H, W, M)
    f = pl.kernel(
        body,
        out_type=jax.ShapeDtypeStruct((9 * NT, L, 128), jnp.float32),
        mesh=mesh,
        scratch_types=[
            pltpu.VMEM((2 * L,), jnp.float32),
            pltpu.VMEM((8, 128), jnp.int32),
            pltpu.VMEM((C * L, L), jnp.float32),
            pltpu.VMEM((C, L), jnp.float32),
            pltpu.VMEM((L, 128), jnp.float32),
            pltpu.SemaphoreType.DMA,
        ],
        compiler_params=pltpu.CompilerParams(needs_layout_passes=False,
                                             use_tc_tiling_on_sc=False),
        name="sc_neighbor_gather",
    )
    return f(kpts_flat, table)


def _tc_reduce(segment, B, C, H, W, HC):
    nh = H // HC

    def body(seg_ref, sum_ref, max_ref):
        h = pl.program_id(1)
        blk = seg_ref[0]
        s = jnp.sum(blk, axis=(1, 2))[None, None, :]
        mx = jnp.max(blk, axis=(1, 2))[None, None, :]

        @pl.when(h == 0)
        def _():
            sum_ref[...] = s
            max_ref[...] = mx

        @pl.when(h != 0)
        def _():
            sum_ref[...] = sum_ref[...] + s
            max_ref[...] = jnp.maximum(max_ref[...], mx)

    return pl.pallas_call(
        body,
        grid=(B, nh),
        in_specs=[pl.BlockSpec((1, C, HC, W), lambda b, h: (b, 0, h, 0))],
        out_specs=[pl.BlockSpec((1, 1, C), lambda b, h: (b, 0, 0)),
                   pl.BlockSpec((1, 1, C), lambda b, h: (b, 0, 0))],
        out_shape=[jax.ShapeDtypeStruct((B, 1, C), jnp.float32),
                   jax.ShapeDtypeStruct((B, 1, C), jnp.float32)],
    )(segment)


def _tc_combine(taps, sums, maxes, conv_w9, conv_b, lin_w, lin_b,
                w1, w2, B, C, H, W, M):
    def body(taps_ref, sum_ref, max_ref, lw_ref, w1_ref, w2_ref,
             cw_ref, cb_ref, lb_ref, out_ref):
        traw = taps_ref[...].reshape(9, B * M, 128)[:, :, :C]
        taps_v = traw.reshape(9, B, M, C)
        cent = taps_v[4]
        pool = cw_ref[0] * taps_v[0]
        for k in range(1, 9):
            pool = pool + cw_ref[k] * taps_v[k]
        pool = pool + cb_ref[0]
        avg = sum_ref[...][:, 0, :] * (1.0 / float(H * W))
        mx = max_ref[...][:, 0, :]
        w1 = w1_ref[...]
        w2 = w2_ref[...]
        a1 = jax.nn.relu(jnp.sum(avg[:, None, :] * w1[None, :, :], axis=2))
        m1 = jax.nn.relu(jnp.sum(mx[:, None, :] * w1[None, :, :], axis=2))
        a2 = jnp.sum(a1[:, None, :] * w2[None, :, :], axis=2)
        m2 = jnp.sum(m1[:, None, :] * w2[None, :, :], axis=2)
        attn = jax.nn.sigmoid(a2 + m2)

        lw = lw_ref[...]
        s_lin = jnp.sum(cent * lw[:, None, :], axis=2) + lb_ref[0]
        norm = jnp.sqrt(jnp.sum(s_lin * s_lin, axis=1, keepdims=True))
        conf = s_lin / jnp.maximum(norm, 1e-12)
        out_ref[...] = ((1.0 - conf)[:, :, None] * pool + cent
                        + attn[:, None, :])

    return pl.pallas_call(
        body,
        in_specs=[
            pl.BlockSpec((9 * B * M, 128), lambda: (0, 0)),
            pl.BlockSpec((B, 1, C), lambda: (0, 0, 0)),
            pl.BlockSpec((B, 1, C), lambda: (0, 0, 0)),
            pl.BlockSpec((1, C), lambda: (0, 0)),
            pl.BlockSpec(w1.shape, lambda: (0, 0)),
            pl.BlockSpec(w2.shape, lambda: (0, 0)),
            pl.BlockSpec(memory_space=pltpu.SMEM),
            pl.BlockSpec(memory_space=pltpu.SMEM),
            pl.BlockSpec(memory_space=pltpu.SMEM),
        ],
        out_specs=pl.BlockSpec((B, M, C), lambda: (0, 0, 0)),
        out_shape=jax.ShapeDtypeStruct((B, M, C), jnp.float32),
    )(taps, sums, maxes, lin_w, w1, w2, conv_w9, conv_b, lin_b)


def kernel(original_kpts, segment, conv_w, conv_b, lin_w, lin_b, ca_fc1,
           ca_fc2):
    B, C, H, W = segment.shape
    M = original_kpts.shape[1]
    HW = H * W

    kpts_flat = original_kpts.reshape(B * M * 2)
    table = (segment
             .reshape(B, C, H // 8, 8, W // 128, 128)
             .transpose(0, 1, 2, 4, 3, 5)
             .reshape(B * C * HW // L, L))

    taps_raw = _sc_gather(kpts_flat, table, B, C, H, W, M)
    sums, maxes = _tc_reduce(segment, B, C, H, W, 192)

    taps4 = taps_raw.reshape(9 * B * M, 128)
    w1 = ca_fc1.reshape(ca_fc1.shape[0], C)
    w2 = ca_fc2.reshape(C, ca_fc2.shape[1])
    return _tc_combine(taps4, sums, maxes, conv_w.reshape(9), conv_b,
                       lin_w, lin_b, w1, w2, B, C, H, W, M)


# speedup vs baseline: 1.0078x; 1.0078x over previous
import functools

import jax
import jax.numpy as jnp
from jax import lax
from jax.experimental import pallas as pl
from jax.experimental.pallas import tpu as pltpu
from jax.experimental.pallas import tpu_sc as plsc

STRIDE9 = 9
L = 16
NT = 32
CCH = 16


def _sc_gather_body(B, C, H, W, M,
                    kpts_hbm, table_hbm, taps_hbm,
                    kv, idx2d, data, tap, tapT, sem):
    HW = H * W
    RPP = HW // L
    ci = lax.axis_index("cx")
    si = lax.axis_index("sx")
    wid = si * 2 + ci
    b = wid // (NT // B)

    pltpu.sync_copy(kpts_hbm.at[pl.ds(wid * 2 * L, 2 * L)], kv)

    lanes = lax.iota(jnp.int32, L)
    kx = plsc.load_gather(kv, [lanes * 2])
    ky = plsc.load_gather(kv, [lanes * 2 + 1])
    x = (kx * float(H)).astype(jnp.int32)
    y = (ky * float(W)).astype(jnp.int32)
    pbase = b * (C * RPP)

    def row_lane(p):
        h = p // W
        w = p - h * W
        row = ((lax.shift_right_logical(h, 3) * (W // 128)
                + lax.shift_right_logical(w, 7)) * 64
               + lax.bitwise_and(h, 7) * 8
               + lax.shift_right_logical(lax.bitwise_and(w, 127), 4))
        return row, lax.bitwise_and(w, 15)

    for k in range(9):
        dx = (k // 3 - 1) * STRIDE9
        dy = (k % 3 - 1) * STRIDE9
        pk = jnp.clip((x + dx) * W + (y + dy), 0, HW - 1)
        rk, lk = row_lane(pk)

        def ibody(cc, _):
            cplane = pbase + cc * RPP
            vk = jnp.full((L,), cplane, jnp.int32) + rk
            idx2d[lax.shift_right_logical(cc, 3),
                  pl.ds(lax.bitwise_and(cc, 7) * L, L)] = vk
            return 0
        lax.fori_loop(0, C, ibody, 0)

        cps = [pltpu.async_copy(table_hbm.at[idx2d.at[j]],
                                data.at[pl.ds(j * 128, 128)], sem)
               for j in range(8)]
        for cp in cps:
            cp.wait()

        def ebody(cc, _):
            rows = cc * L + lanes
            val = plsc.load_gather(data, [rows, lk])
            tap[cc, :] = val
            return 0
        lax.fori_loop(0, C, ebody, 0)

        def tbody(c, _):
            cvec = jnp.full((L,), c, jnp.int32)
            plsc.store_scatter(tapT, [lanes, cvec], tap[c, :])
            return 0
        lax.fori_loop(0, C, tbody, 0)
        pltpu.sync_copy(tapT, taps_hbm.at[k * NT + wid])


def _sc_gather(kpts_flat, table, B, C, H, W, M):
    mesh = plsc.VectorSubcoreMesh(core_axis_name="cx", subcore_axis_name="sx")
    body = functools.partial(_sc_gather_body, B, C, H, W, M)
    f = pl.kernel(
        body,
        out_type=jax.ShapeDtypeStruct((9 * NT, L, 128), jnp.float32),
        mesh=mesh,
        scratch_types=[
            pltpu.VMEM((2 * L,), jnp.float32),
            pltpu.VMEM((8, 128), jnp.int32),
            pltpu.VMEM((C * L, L), jnp.float32),
            pltpu.VMEM((C, L), jnp.float32),
            pltpu.VMEM((L, 128), jnp.float32),
            pltpu.SemaphoreType.DMA,
        ],
        compiler_params=pltpu.CompilerParams(needs_layout_passes=False,
                                             use_tc_tiling_on_sc=False),
        name="sc_neighbor_gather",
    )
    return f(kpts_flat, table)


def _tc_reduce(segment, B, C, H, W, HC):
    nh = H // HC

    def body(seg_ref, sum_ref, max_ref):
        h = pl.program_id(1)
        blk = seg_ref[0]
        s = jnp.sum(blk, axis=(1, 2))[None, None, :]
        mx = jnp.max(blk, axis=(1, 2))[None, None, :]

        @pl.when(h == 0)
        def _():
            sum_ref[...] = s
            max_ref[...] = mx

        @pl.when(h != 0)
        def _():
            sum_ref[...] = sum_ref[...] + s
            max_ref[...] = jnp.maximum(max_ref[...], mx)

    return pl.pallas_call(
        body,
        grid=(B, nh),
        in_specs=[pl.BlockSpec((1, C, HC, W), lambda b, h: (b, 0, h, 0))],
        out_specs=[pl.BlockSpec((1, 1, C), lambda b, h: (b, 0, 0)),
                   pl.BlockSpec((1, 1, C), lambda b, h: (b, 0, 0))],
        out_shape=[jax.ShapeDtypeStruct((B, 1, C), jnp.float32),
                   jax.ShapeDtypeStruct((B, 1, C), jnp.float32)],
    )(segment)


def _tc_combine(taps, sums, maxes, conv_w9, conv_b, lin_w, lin_b,
                w1, w2, B, C, H, W, M):
    def body(taps_ref, sum_ref, max_ref, lw_ref, w1_ref, w2_ref,
             cw_ref, cb_ref, lb_ref, out_ref):
        traw = taps_ref[...].reshape(9, B * M, 128)[:, :, :C]
        taps_v = traw.reshape(9, B, M, C)
        cent = taps_v[4]
        pool = cw_ref[0] * taps_v[0]
        for k in range(1, 9):
            pool = pool + cw_ref[k] * taps_v[k]
        pool = pool + cb_ref[0]
        avg = sum_ref[...][:, 0, :] * (1.0 / float(H * W))
        mx = max_ref[...][:, 0, :]
        w1 = w1_ref[...]
        w2 = w2_ref[...]
        a1 = jax.nn.relu(jnp.sum(avg[:, None, :] * w1[None, :, :], axis=2))
        m1 = jax.nn.relu(jnp.sum(mx[:, None, :] * w1[None, :, :], axis=2))
        a2 = jnp.sum(a1[:, None, :] * w2[None, :, :], axis=2)
        m2 = jnp.sum(m1[:, None, :] * w2[None, :, :], axis=2)
        attn = jax.nn.sigmoid(a2 + m2)

        lw = lw_ref[...]
        s_lin = jnp.sum(cent * lw[:, None, :], axis=2) + lb_ref[0]
        norm = jnp.sqrt(jnp.sum(s_lin * s_lin, axis=1, keepdims=True))
        conf = s_lin / jnp.maximum(norm, 1e-12)
        out_ref[...] = ((1.0 - conf)[:, :, None] * pool + cent
                        + attn[:, None, :])

    return pl.pallas_call(
        body,
        in_specs=[
            pl.BlockSpec((9 * B * M, 128), lambda: (0, 0)),
            pl.BlockSpec((B, 1, C), lambda: (0, 0, 0)),
            pl.BlockSpec((B, 1, C), lambda: (0, 0, 0)),
            pl.BlockSpec((1, C), lambda: (0, 0)),
            pl.BlockSpec(w1.shape, lambda: (0, 0)),
            pl.BlockSpec(w2.shape, lambda: (0, 0)),
            pl.BlockSpec(memory_space=pltpu.SMEM),
            pl.BlockSpec(memory_space=pltpu.SMEM),
            pl.BlockSpec(memory_space=pltpu.SMEM),
        ],
        out_specs=pl.BlockSpec((B, M, C), lambda: (0, 0, 0)),
        out_shape=jax.ShapeDtypeStruct((B, M, C), jnp.float32),
    )(taps, sums, maxes, lin_w, w1, w2, conv_w9, conv_b, lin_b)


def kernel(original_kpts, segment, conv_w, conv_b, lin_w, lin_b, ca_fc1,
           ca_fc2):
    B, C, H, W = segment.shape
    M = original_kpts.shape[1]
    HW = H * W

    kpts_flat = original_kpts.reshape(B * M * 2)
    table = (segment
             .reshape(B, C, H // 8, 8, W // 128, 128)
             .transpose(0, 1, 2, 4, 3, 5)
             .reshape(B * C * HW // L, L))

    taps_raw = _sc_gather(kpts_flat, table, B, C, H, W, M)
    sums, maxes = _tc_reduce(segment, B, C, H, W, 128)

    taps4 = taps_raw.reshape(9 * B * M, 128)
    w1 = ca_fc1.reshape(ca_fc1.shape[0], C)
    w2 = ca_fc2.reshape(C, ca_fc2.shape[1])
    return _tc_combine(taps4, sums, maxes, conv_w.reshape(9), conv_b,
                       lin_w, lin_b, w1, w2, B, C, H, W, M)
